# baseline (speedup 1.0000x reference)
import functools

import jax
import jax.numpy as jnp
from jax import lax
from jax.experimental import pallas as pl
from jax.experimental.pallas import tpu as pltpu
from jax.experimental.pallas import tpu_sc as plsc

DIM = 2048
N_EXPERTS = 64
TOPK = 8
N_GROUPS = 8
GROUP_SIZE = N_EXPERTS // N_GROUPS
TOPK_GROUPS = 4
BB = 2048

NW = 32
RPW = 16384 // NW
CHUNK = 16
NCAND = TOPK_GROUPS * GROUP_SIZE


def _tc_body(x_ref, w_ref, pm_ref, gid_ref):
    x = x_ref[...]
    w = w_ref[...]
    s = jax.lax.dot_general(
        w, x, (((1,), (1,)), ((), ())), preferred_element_type=jnp.float32
    )
    m = jnp.max(s, axis=0, keepdims=True)
    e = jnp.exp(s - m)
    p = e / jnp.sum(e, axis=0, keepdims=True)

    g = jnp.concatenate(
        [
            jnp.max(p[GROUP_SIZE * i : GROUP_SIZE * (i + 1)], axis=0, keepdims=True)
            for i in range(N_GROUPS)
        ],
        axis=0,
    )

    iota_g = jax.lax.broadcasted_iota(jnp.int32, g.shape, 0)
    gmask = jnp.zeros_like(g)
    gids = []
    for _ in range(TOPK_GROUPS):
        mg = jnp.max(g, axis=0, keepdims=True)
        amg = jnp.min(jnp.where(g == mg, iota_g, N_GROUPS), axis=0, keepdims=True)
        sel = iota_g == amg
        gmask = jnp.where(sel, 1.0, gmask)
        g = jnp.where(sel, -1.0, g)
        gids.append(amg)

    pm_ref[...] = jnp.concatenate(
        [
            p[GROUP_SIZE * i : GROUP_SIZE * (i + 1)] * gmask[i : i + 1]
            for i in range(N_GROUPS)
        ],
        axis=0,
    )
    gid_ref[...] = jnp.concatenate(gids, axis=0)


def _tc_stage(x, weight):
    B = x.shape[0]
    return pl.pallas_call(
        _tc_body,
        grid=(B // BB,),
        in_specs=[
            pl.BlockSpec((BB, DIM), lambda i: (i, 0)),
            pl.BlockSpec((N_EXPERTS, DIM), lambda i: (0, 0)),
        ],
        out_specs=[
            pl.BlockSpec((N_EXPERTS, BB), lambda i: (0, i)),
            pl.BlockSpec((TOPK_GROUPS, BB), lambda i: (0, i)),
        ],
        out_shape=[
            jax.ShapeDtypeStruct((N_EXPERTS, B), jnp.float32),
            jax.ShapeDtypeStruct((TOPK_GROUPS, B), jnp.int32),
        ],
        compiler_params=pltpu.CompilerParams(
            dimension_semantics=("arbitrary",),
        ),
    )(x, weight)


def _sc_topk(pm_t, gid_t):
    B = pm_t.shape[1]
    mesh = plsc.VectorSubcoreMesh(core_axis_name="c", subcore_axis_name="s")

    @functools.partial(
        pl.kernel,
        mesh=mesh,
        out_type=[
            jax.ShapeDtypeStruct((TOPK, B), jnp.float32),
            jax.ShapeDtypeStruct((TOPK, B), jnp.int32),
        ],
        scratch_types=[
            pltpu.VMEM((N_EXPERTS * RPW,), jnp.float32),
            pltpu.VMEM((TOPK_GROUPS * RPW,), jnp.int32),
            pltpu.VMEM((TOPK * RPW,), jnp.float32),
            pltpu.VMEM((TOPK * RPW,), jnp.int32),
        ],
    )
    def k(pm_hbm, gid_hbm, wout_hbm, iout_hbm, ptile, gtile, wtile, itile):
        wid = lax.axis_index("s") * 2 + lax.axis_index("c")
        base = wid * RPW
        for e in range(N_EXPERTS):
            pltpu.sync_copy(
                pm_hbm.at[e, pl.ds(base, RPW)], ptile.at[pl.ds(e * RPW, RPW)]
            )
        for i in range(TOPK_GROUPS):
            pltpu.sync_copy(
                gid_hbm.at[i, pl.ds(base, RPW)], gtile.at[pl.ds(i * RPW, RPW)]
            )

        def chunk_body(j):
            off = j * CHUNK
            gbase = []
            cand = []
            zero = jnp.zeros((16,), jnp.float32)
            for i in range(TOPK_GROUPS):
                gid = gtile[pl.ds(i * RPW + off, CHUNK)]
                gbase.append(gid * GROUP_SIZE)
                gm = [
                    gid == jnp.full((16,), g_, jnp.int32) for g_ in range(N_GROUPS)
                ]
                for r in range(GROUP_SIZE):
                    v = zero
                    for g_ in range(N_GROUPS):
                        v = jnp.where(
                            gm[g_],
                            ptile[pl.ds((g_ * GROUP_SIZE + r) * RPW + off, CHUNK)],
                            v,
                        )
                    cand.append(v)
            for k_ in range(TOPK):
                m = cand[0]
                for c in range(1, NCAND):
                    m = jnp.maximum(m, cand[c])
                ai = jnp.full((16,), N_EXPERTS, jnp.int32)
                for c in range(NCAND):
                    cidx = gbase[c // GROUP_SIZE] + (c % GROUP_SIZE)
                    ai = jnp.minimum(
                        ai,
                        jnp.where(cand[c] == m, cidx,
                                  jnp.full((16,), N_EXPERTS, jnp.int32)),
                    )
                for c in range(NCAND):
                    cidx = gbase[c // GROUP_SIZE] + (c % GROUP_SIZE)
                    cand[c] = jnp.where(
                        cidx == ai, jnp.full((16,), -1.0, jnp.float32), cand[c]
                    )
                wtile[pl.ds(k_ * RPW + off, CHUNK)] = jnp.maximum(
                    m, jnp.full((16,), 1e-7, jnp.float32)
                )
                itile[pl.ds(k_ * RPW + off, CHUNK)] = ai

        lax.fori_loop(0, RPW // CHUNK, lambda j, c: (chunk_body(j), c)[1], 0)

        for k_ in range(TOPK):
            pltpu.sync_copy(
                wtile.at[pl.ds(k_ * RPW, RPW)], wout_hbm.at[k_, pl.ds(base, RPW)]
            )
            pltpu.sync_copy(
                itile.at[pl.ds(k_ * RPW, RPW)], iout_hbm.at[k_, pl.ds(base, RPW)]
            )

    return k(pm_t, gid_t)


@functools.partial(jax.jit, static_argnames=())
def kernel(x, weight):
    pm_t, gid_t = _tc_stage(x, weight)
    w_t, i_t = _sc_topk(pm_t, gid_t)
    return w_t.T, i_t.T
